# baseline (speedup 1.0000x reference)
import functools

import jax
import jax.numpy as jnp
from jax import lax
from jax.experimental import pallas as pl
from jax.experimental.pallas import tpu as pltpu
from jax.experimental.pallas import tpu_sc as plsc

BATCH = 4096
HIST = 50
HIST_PAD = 56
EMBED = 128
NUM_WORKERS = 32
ROWS_PER_W = BATCH // NUM_WORKERS
K = 8
NGROUP = ROWS_PER_W // K
NBUF = 2

_mesh = plsc.VectorSubcoreMesh(core_axis_name="c", subcore_axis_name="s")


@functools.partial(
    pl.kernel,
    out_type=jax.ShapeDtypeStruct((BATCH, HIST_PAD, EMBED), jnp.float32),
    mesh=_mesh,
    scratch_types=[
        pltpu.VMEM((ROWS_PER_W, HIST), jnp.int32),
        pltpu.VMEM((NBUF, K, HIST_PAD, EMBED), jnp.float32),
        pltpu.SemaphoreType.DMA,
        pltpu.SemaphoreType.DMA,
    ],
)
def _emb_gather(idx_hbm, table_hbm, out_hbm, idx_v, rows_v, gsem, ssem):
    wid = lax.axis_index("s") * 2 + lax.axis_index("c")
    base = wid * ROWS_PER_W
    pltpu.sync_copy(idx_hbm.at[pl.ds(base, ROWS_PER_W)], idx_v)

    def g_copy(g, k):
        return pltpu.make_async_copy(
            table_hbm.at[idx_v.at[g * K + k]],
            rows_v.at[g % NBUF, k, pl.ds(0, HIST)], gsem)

    def s_copy(g):
        return pltpu.make_async_copy(
            rows_v.at[g % NBUF],
            out_hbm.at[pl.ds(base + g * K, K)], ssem)

    def start_group(g):
        for k in range(K):
            g_copy(g, k).start()

    def wait_group(g):
        for k in range(K):
            g_copy(g, k).wait()

    start_group(0)

    @pl.loop(0, NGROUP)
    def _body(g):
        @pl.when(g > 0)
        def _():
            s_copy(g - 1).wait()

        @pl.when(g + 1 < NGROUP)
        def _():
            start_group(g + 1)

        wait_group(g)
        s_copy(g).start()

    s_copy(NGROUP - 1).wait()


def kernel(x, table):
    out = _emb_gather(x.astype(jnp.int32), table)
    return out[:, :HIST, :]


# speedup vs baseline: 1.1689x; 1.1689x over previous
import functools

import jax
import jax.numpy as jnp
from jax import lax
from jax.experimental import pallas as pl
from jax.experimental.pallas import tpu as pltpu
from jax.experimental.pallas import tpu_sc as plsc

BATCH = 4096
HIST = 50
EMBED = 128
NUM_WORKERS = 32
ROWS_PER_W = BATCH // NUM_WORKERS
K = 8
NGROUP = ROWS_PER_W // K
NBUF = 2

_mesh = plsc.VectorSubcoreMesh(core_axis_name="c", subcore_axis_name="s")


@functools.partial(
    pl.kernel,
    out_type=jax.ShapeDtypeStruct((BATCH, HIST, EMBED), jnp.float32),
    mesh=_mesh,
    scratch_types=[
        pltpu.VMEM((ROWS_PER_W, HIST), jnp.int32),
        pltpu.VMEM((NBUF, K, HIST, EMBED), jnp.float32),
        pltpu.SemaphoreType.DMA,
        pltpu.SemaphoreType.DMA,
    ],
    compiler_params=pltpu.CompilerParams(needs_layout_passes=True),
)
def _emb_gather(idx_hbm, table_hbm, out_hbm, idx_v, rows_v, gsem, ssem):
    wid = lax.axis_index("s") * 2 + lax.axis_index("c")
    base = wid * ROWS_PER_W
    pltpu.sync_copy(idx_hbm.at[pl.ds(base, ROWS_PER_W)], idx_v)

    def g_copy(g, k):
        return pltpu.make_async_copy(
            table_hbm.at[idx_v.at[g * K + k]],
            rows_v.at[g % NBUF, k], gsem)

    def s_copy(g):
        return pltpu.make_async_copy(
            rows_v.at[g % NBUF],
            out_hbm.at[pl.ds(base + g * K, K)], ssem)

    def start_group(g):
        for k in range(K):
            g_copy(g, k).start()

    def wait_group(g):
        for k in range(K):
            g_copy(g, k).wait()

    start_group(0)

    @pl.loop(0, NGROUP)
    def _body(g):
        @pl.when(g > 0)
        def _():
            s_copy(g - 1).wait()

        @pl.when(g + 1 < NGROUP)
        def _():
            start_group(g + 1)

        wait_group(g)
        s_copy(g).start()

    s_copy(NGROUP - 1).wait()


def kernel(x, table):
    return _emb_gather(x.astype(jnp.int32), table)
